# baseline (speedup 1.0000x reference)
import functools

import jax
import jax.numpy as jnp
from jax import lax
from jax.experimental import pallas as pl
from jax.experimental.pallas import tpu as pltpu
from jax.experimental.pallas import tpu_sc as plsc

_E = 16
_K = 2
_D = 2048
_TB = 1024
_NW = 32
_L = 16


def _logits_block(x_ref, wt_ref, b_ref, logits_ref):
    logits = jnp.dot(x_ref[...], wt_ref[...],
                     preferred_element_type=jnp.float32) + b_ref[...]
    logits_ref[...] = logits.T


def _sc_router(logits_hbm, probs_hbm, wts_hbm, idx_hbm, lt_v, pt_v, wts_v, idx_v):
    tpw = logits_hbm.shape[1] // _NW
    wid = lax.axis_index("s") * 2 + lax.axis_index("c")
    base = wid * tpw
    pltpu.sync_copy(logits_hbm.at[:, pl.ds(base, tpw)], lt_v)

    def group(g, _):
        ds = pl.ds(g * _L, _L)
        l = [lt_v[e, ds] for e in range(_E)]
        m = l[0]
        for e in range(1, _E):
            m = jnp.maximum(m, l[e])
        ex = [jnp.exp(l[e] - m) for e in range(_E)]
        s = ex[0]
        for e in range(1, _E):
            s = s + ex[e]
        r = 1.0 / s
        p = [ex[e] * r for e in range(_E)]
        for e in range(_E):
            pt_v[e, ds] = p[e]

        m1 = p[0]
        for e in range(1, _E):
            m1 = jnp.maximum(m1, p[e])
        i1 = jnp.full((_L,), _E, jnp.int32)
        for e in range(_E - 1, -1, -1):
            i1 = jnp.where(p[e] == m1, jnp.full((_L,), e, jnp.int32), i1)
        ninf = jnp.full((_L,), -jnp.inf, jnp.float32)
        m2 = ninf
        for e in range(_E):
            m2 = jnp.maximum(m2, jnp.where(i1 == e, ninf, p[e]))
        i2 = jnp.full((_L,), _E, jnp.int32)
        for e in range(_E - 1, -1, -1):
            i2 = jnp.where((p[e] == m2) & (i1 != e),
                           jnp.full((_L,), e, jnp.int32), i2)
        rs = 1.0 / (m1 + m2)
        wts_v[0, ds] = m1 * rs
        wts_v[1, ds] = m2 * rs
        idx_v[0, ds] = i1
        idx_v[1, ds] = i2
        return ()

    lax.fori_loop(0, tpw // _L, group, (), unroll=False)

    pltpu.sync_copy(pt_v, probs_hbm.at[:, pl.ds(base, tpw)])
    pltpu.sync_copy(wts_v, wts_hbm.at[:, pl.ds(base, tpw)])
    pltpu.sync_copy(idx_v, idx_hbm.at[:, pl.ds(base, tpw)])


@functools.partial(jax.jit, static_argnames=("interpret",))
def kernel(inputs, W, b, interpret=False):
    B, S, D = inputs.shape
    T = B * S
    x = inputs.reshape(T, D)
    wt = W.T
    b2 = b.reshape(1, _E)

    logits_t = pl.pallas_call(
        _logits_block,
        grid=(T // _TB,),
        in_specs=[
            pl.BlockSpec((_TB, D), lambda i: (i, 0)),
            pl.BlockSpec((D, _E), lambda i: (0, 0)),
            pl.BlockSpec((1, _E), lambda i: (0, 0)),
        ],
        out_specs=pl.BlockSpec((_E, _TB), lambda i: (0, i)),
        out_shape=jax.ShapeDtypeStruct((_E, T), jnp.float32),
        compiler_params=pltpu.CompilerParams(
            dimension_semantics=("parallel",),
        ),
        interpret=interpret,
    )(x, wt, b2)

    tpw = T // _NW
    mesh = plsc.VectorSubcoreMesh(core_axis_name="c", subcore_axis_name="s")
    probs_t, wts_t, idx_t = pl.kernel(
        _sc_router,
        out_type=[
            jax.ShapeDtypeStruct((_E, T), jnp.float32),
            jax.ShapeDtypeStruct((_K, T), jnp.float32),
            jax.ShapeDtypeStruct((_K, T), jnp.int32),
        ],
        mesh=mesh,
        scratch_types=[
            pltpu.VMEM((_E, tpw), jnp.float32),
            pltpu.VMEM((_E, tpw), jnp.float32),
            pltpu.VMEM((_K, tpw), jnp.float32),
            pltpu.VMEM((_K, tpw), jnp.int32),
        ],
    )(logits_t)

    return (logits_t.T.reshape(B, S, _E), probs_t.T.reshape(B, S, _E),
            wts_t.T.reshape(B, S, _K), idx_t.T.reshape(B, S, _K))


# speedup vs baseline: 1.4177x; 1.4177x over previous
import functools

import jax
import jax.numpy as jnp
from jax.experimental import pallas as pl
from jax.experimental.pallas import tpu as pltpu

_E = 16
_K = 2
_D = 2048
_TB = 1024


def _router_block(x_ref, wt_ref, b_ref,
                  logits_ref, probs_ref, wts_ref, idx_ref):
    logits = jnp.dot(x_ref[...], wt_ref[...],
                     preferred_element_type=jnp.float32) + b_ref[...]
    lt = logits.T
    logits_ref[...] = lt

    m = jnp.max(lt, axis=0, keepdims=True)
    e = jnp.exp(lt - m)
    pt = e / jnp.sum(e, axis=0, keepdims=True)
    probs_ref[...] = pt

    iota = jax.lax.broadcasted_iota(jnp.int32, pt.shape, 0)
    m1 = jnp.max(pt, axis=0, keepdims=True)
    i1 = jnp.min(jnp.where(pt == m1, iota, _E), axis=0, keepdims=True)
    masked = jnp.where(iota == i1, -jnp.inf, pt)
    m2 = jnp.max(masked, axis=0, keepdims=True)
    i2 = jnp.min(jnp.where(masked == m2, iota, _E), axis=0, keepdims=True)

    s = m1 + m2
    wts_ref[...] = jnp.concatenate([m1 / s, m2 / s], axis=0)
    idx_ref[...] = jnp.concatenate([i1, i2], axis=0)


@functools.partial(jax.jit, static_argnames=("interpret",))
def kernel(inputs, W, b, interpret=False):
    B, S, D = inputs.shape
    T = B * S
    x = inputs.reshape(T, D)
    wt = W.T
    b2 = b.reshape(1, _E)

    logits_t, probs_t, wts_t, idx_t = pl.pallas_call(
        _router_block,
        grid=(T // _TB,),
        in_specs=[
            pl.BlockSpec((_TB, D), lambda i: (i, 0)),
            pl.BlockSpec((D, _E), lambda i: (0, 0)),
            pl.BlockSpec((1, _E), lambda i: (0, 0)),
        ],
        out_specs=[
            pl.BlockSpec((_E, _TB), lambda i: (0, i)),
            pl.BlockSpec((_E, _TB), lambda i: (0, i)),
            pl.BlockSpec((_K, _TB), lambda i: (0, i)),
            pl.BlockSpec((_K, _TB), lambda i: (0, i)),
        ],
        out_shape=[
            jax.ShapeDtypeStruct((_E, T), jnp.float32),
            jax.ShapeDtypeStruct((_E, T), jnp.float32),
            jax.ShapeDtypeStruct((_K, T), jnp.float32),
            jax.ShapeDtypeStruct((_K, T), jnp.int32),
        ],
        compiler_params=pltpu.CompilerParams(
            dimension_semantics=("parallel",),
        ),
        interpret=interpret,
    )(x, wt, b2)

    return (logits_t.T.reshape(B, S, _E), probs_t.T.reshape(B, S, _E),
            wts_t.T.reshape(B, S, _K), idx_t.T.reshape(B, S, _K))
